# baseline (speedup 1.0000x reference)
import jax
import jax.numpy as jnp
from jax.experimental import pallas as pl

EPS = 1e-11

ROWS = 128
COLS = 100000
BLOCK_ROWS = 8


def _gumbel_softmax_kernel(x_ref, u_ref, o_ref):
    x = x_ref[...]
    u = u_ref[...]
    g = -jnp.log(-jnp.log(u + EPS))
    y = x + g
    m = jnp.max(y, axis=-1, keepdims=True)
    e = jnp.exp(y - m)
    s = jnp.sum(e, axis=-1, keepdims=True)
    o_ref[...] = e * (1.0 / s)


def kernel(logits, u):
    grid = (ROWS // BLOCK_ROWS,)
    spec = pl.BlockSpec((BLOCK_ROWS, COLS), lambda i: (i, 0))
    return pl.pallas_call(
        _gumbel_softmax_kernel,
        grid=grid,
        in_specs=[spec, spec],
        out_specs=spec,
        out_shape=jax.ShapeDtypeStruct((ROWS, COLS), jnp.float32),
    )(logits, u)
